# baseline (speedup 1.0000x reference)
import functools

import jax
import jax.numpy as jnp
from jax import lax
from jax.experimental import pallas as pl
from jax.experimental.pallas import tpu as pltpu
from jax.experimental.pallas import tpu_sc as plsc

_N = 10000
_E = 320000
_EP = 327680
_H = 128
_NW = 32
_EPW = _EP // _NW
_CH = 128
_NCH = _EPW // _CH
_NP = 10240
_RPT = _NP // 16
_EB = 4096
_NEB = _EP // _EB


def _silu(x):
    return x * jax.nn.sigmoid(x)


def _onehot(idx_col, k):
    rows = idx_col.shape[0]
    return (lax.broadcasted_iota(jnp.int32, (rows, k), 1) == idx_col).astype(
        jnp.float32)


def _dot(a, b):
    return jnp.dot(a, b, preferred_element_type=jnp.float32)


def _tables(h, oh_r, oh_c, ws_ref, wd_ref, w32_ref, remb_ref, cemb_ref,
            relp_ref, wrel_ref, eb1_ref):
    w32 = w32_ref[...]
    a = (_dot(h, ws_ref[...])
         + _dot(oh_r, _dot(remb_ref[...], w32[0:8]))
         + _dot(oh_c, _dot(cemb_ref[...], w32[16:24])))
    b = (_dot(h, wd_ref[...])
         + _dot(oh_r, _dot(remb_ref[...], w32[8:16]))
         + _dot(oh_c, _dot(cemb_ref[...], w32[24:32]))
         + eb1_ref[...])
    relc = _dot(relp_ref[...], wrel_ref[...])
    return a, b, relc


def _encode_body(scalars_ref, color_ref, role_ref, w0_ref, wc_ref, wr_ref,
                 bin_ref, bcemb_ref, bremb_ref,
                 ws_ref, wd_ref, w32_ref, remb_ref, cemb_ref, relp_ref,
                 wrel_ref, eb1_ref,
                 h_ref, a_ref, b_ref, relc_ref):
    oh_c = _onehot(color_ref[...], 8)
    oh_r = _onehot(role_ref[...], 8)
    h = (_dot(scalars_ref[...], w0_ref[...])
         + _dot(oh_c, _dot(bcemb_ref[...], wc_ref[...]))
         + _dot(oh_r, _dot(bremb_ref[...], wr_ref[...]))
         + bin_ref[...])
    h_ref[...] = h
    a, b, relc = _tables(h, oh_r, oh_c, ws_ref, wd_ref, w32_ref, remb_ref,
                         cemb_ref, relp_ref, wrel_ref, eb1_ref)
    a_ref[...] = a
    b_ref[...] = b
    relc_ref[...] = relc


def _node_body(last, h_ref, agg0_ref, agg1_ref, color_ref, role_ref,
               nwa_ref, nwb_ref, nw16_ref, nremb_ref, ncemb_ref, nb1_ref,
               nw2_ref, nb2_ref, lng_ref, lnb_ref, *rest):
    oh_c = _onehot(color_ref[...], 8)
    oh_r = _onehot(role_ref[...], 8)
    h = h_ref[...]
    agg = agg0_ref[...] + agg1_ref[...]
    nw16 = nw16_ref[...]
    node_pre = (_dot(h, nwa_ref[...]) + _dot(agg, nwb_ref[...])
                + _dot(oh_r, _dot(nremb_ref[...], nw16[0:8]))
                + _dot(oh_c, _dot(ncemb_ref[...], nw16[8:16]))
                + nb1_ref[...])
    update = _dot(_silu(node_pre), nw2_ref[...]) + nb2_ref[...]
    hr = h + update
    mu = jnp.mean(hr, axis=-1, keepdims=True)
    d = hr - mu
    var = jnp.mean(d * d, axis=-1, keepdims=True)
    hn = d * lax.rsqrt(var + 1e-5) * lng_ref[...] + lnb_ref[...]
    if last:
        (wout_ref, bout_ref, out_ref) = rest
        out_ref[...] = _dot(hn, wout_ref[...]) + bout_ref[...]
    else:
        (ws_ref, wd_ref, w32_ref, remb_ref, cemb_ref, relp_ref, wrel_ref,
         eb1_ref, hn_ref, a_ref, b_ref, relc_ref) = rest
        hn_ref[...] = hn
        a, b, relc = _tables(hn, oh_r, oh_c, ws_ref, wd_ref, w32_ref,
                             remb_ref, cemb_ref, relp_ref, wrel_ref, eb1_ref)
        a_ref[...] = a
        b_ref[...] = b
        relc_ref[...] = relc


def _edge_body(pre0_ref, rel_ref, relc_ref, ew2_ref, eb2_ref, m_ref):
    rel = rel_ref[0]
    oh = (lax.broadcasted_iota(jnp.int32, (_EB, 16), 1) == rel).astype(
        jnp.float32)
    pre1 = pre0_ref[...] + _dot(oh, relc_ref[...])
    t2 = _dot(_silu(pre1), ew2_ref[...]) + eb2_ref[...]
    m_ref[...] = _silu(t2)


def _edge_call(pre0, rel3, relc, ew2, eb2):
    return pl.pallas_call(
        _edge_body,
        grid=(_NEB,),
        in_specs=[
            pl.BlockSpec((_EB, _H), lambda i: (i, 0)),
            pl.BlockSpec((1, _EB, 1), lambda i: (i, 0, 0)),
            pl.BlockSpec((16, _H), lambda i: (0, 0)),
            pl.BlockSpec((_H, _H), lambda i: (0, 0)),
            pl.BlockSpec((1, _H), lambda i: (0, 0)),
        ],
        out_specs=pl.BlockSpec((_EB, _H), lambda i: (i, 0)),
        out_shape=jax.ShapeDtypeStruct((_EP, _H), jnp.float32),
    )(pre0, rel3, relc, ew2, eb2)


def _vadd_rows(dst_ref, src_ref):
    def row(r, carry):
        for p in range(8):
            sl = pl.ds(p * 16, 16)
            dst_ref[r, sl] = dst_ref[r, sl] + src_ref[r, sl]
        return carry

    lax.fori_loop(0, _CH, row, 0, unroll=False)


@functools.lru_cache(maxsize=None)
def _sc_kernels():
    mesh = plsc.VectorSubcoreMesh(core_axis_name="c", subcore_axis_name="s")

    @functools.partial(
        pl.kernel,
        out_type=jax.ShapeDtypeStruct((_EP, _H), jnp.float32),
        mesh=mesh,
        scratch_types=[
            pltpu.VMEM((_NCH, _CH), jnp.int32),
            pltpu.VMEM((_NCH, _CH), jnp.int32),
            pltpu.VMEM((_CH, _H), jnp.float32),
            pltpu.VMEM((_CH, _H), jnp.float32),
            pltpu.VMEM((_CH, _H), jnp.float32),
            pltpu.VMEM((_CH, _H), jnp.float32),
            pltpu.SemaphoreType.DMA,
            pltpu.SemaphoreType.DMA,
            pltpu.SemaphoreType.DMA,
            pltpu.SemaphoreType.DMA,
        ],
    )
    def _sc_gather(a_hbm, b_hbm, src3_hbm, dst3_hbm, out_hbm,
                   idxs_v, idxd_v, bufa0, bufb0, bufa1, bufb1,
                   ga0, gb0, ga1, gb1):
        wid = lax.axis_index("s") * 2 + lax.axis_index("c")
        wbase = wid * _EPW
        pltpu.sync_copy(src3_hbm.at[wid], idxs_v)
        pltpu.sync_copy(dst3_hbm.at[wid], idxd_v)

        def issue(j, bufa, bufb, ga, gb):
            pltpu.async_copy(a_hbm.at[idxs_v.at[j]], bufa, ga)
            pltpu.async_copy(b_hbm.at[idxd_v.at[j]], bufb, gb)

        def wait(buf, sem):
            pltpu.make_async_copy(a_hbm.at[pl.ds(0, _CH)], buf, sem).wait()

        def finish(j, bufa, bufb, ga, gb):
            wait(bufa, ga)
            wait(bufb, gb)
            _vadd_rows(bufa, bufb)
            pltpu.sync_copy(bufa, out_hbm.at[pl.ds(wbase + j * _CH, _CH)])

        issue(0, bufa0, bufb0, ga0, gb0)

        def body(i, carry):
            j0 = 2 * i
            issue(j0 + 1, bufa1, bufb1, ga1, gb1)
            finish(j0, bufa0, bufb0, ga0, gb0)

            @pl.when(i < _NCH // 2 - 1)
            def _():
                issue(j0 + 2, bufa0, bufb0, ga0, gb0)

            finish(j0 + 1, bufa1, bufb1, ga1, gb1)
            return carry

        lax.fori_loop(0, _NCH // 2, body, 0, unroll=False)

    @functools.partial(
        pl.kernel,
        out_type=jax.ShapeDtypeStruct((2, _NP, _H), jnp.float32),
        mesh=mesh,
        scratch_types=[
            pltpu.VMEM_SHARED((_NP, _H), jnp.float32),
            pltpu.VMEM((_NCH, _CH), jnp.int32),
            pltpu.VMEM((_CH, _H), jnp.float32),
            pltpu.VMEM((_CH, _H), jnp.float32),
            pltpu.SemaphoreType.DMA,
            pltpu.SemaphoreType.DMA,
        ],
    )
    def _sc_scatter(m_hbm, dst3_hbm, out_hbm, agg_sh, idx_v, m0, m1, r0, r1):
        cid = lax.axis_index("c")
        sid = lax.axis_index("s")
        wid = sid * 2 + cid
        wbase = wid * _EPW
        pltpu.sync_copy(dst3_hbm.at[wid], idx_v)

        def zrow(r, carry):
            for p in range(8):
                m0[r, pl.ds(p * 16, 16)] = jnp.zeros((16,), jnp.float32)
            return carry

        lax.fori_loop(0, _CH, zrow, 0, unroll=False)
        for k in range(5):
            pltpu.sync_copy(m0, agg_sh.at[pl.ds(sid * _RPT + k * _CH, _CH)])
        plsc.subcore_barrier()

        def issue(j, buf, sem):
            pltpu.async_copy(m_hbm.at[pl.ds(wbase + j * _CH, _CH)], buf, sem)

        def finish(j, buf, sem):
            pltpu.make_async_copy(m_hbm.at[pl.ds(0, _CH)], buf, sem).wait()
            pltpu.sync_copy(buf, agg_sh.at[idx_v.at[j]], add=True)

        issue(0, m0, r0)

        def body(i, carry):
            j0 = 2 * i
            issue(j0 + 1, m1, r1)
            finish(j0, m0, r0)

            @pl.when(i < _NCH // 2 - 1)
            def _():
                issue(j0 + 2, m0, r0)

            finish(j0 + 1, m1, r1)
            return carry

        lax.fori_loop(0, _NCH // 2, body, 0, unroll=False)
        plsc.subcore_barrier()
        pltpu.sync_copy(agg_sh.at[pl.ds(sid * _RPT, _RPT)],
                        out_hbm.at[cid, pl.ds(sid * _RPT, _RPT)])

    return _sc_gather, _sc_scatter


def _pad_rows(x, rows):
    return jnp.pad(x, ((0, rows - x.shape[0]), (0, 0)))


def kernel(scalars, blk_color_emb, blk_role_emb, w_in, b_in, rel_emb,
           lyr_role_emb, lyr_color_emb, ew1, eb1, ew2, eb2, nw1, nb1, nw2,
           nb2, ln_g, ln_b, w_out, b_out, edge_index, edge_relation,
           node_color_rep, node_role):
    pad = _EP - _E
    src3 = jnp.concatenate(
        [edge_index[0], jnp.zeros((pad,), jnp.int32)]).reshape(_NW, _NCH, _CH)
    dst3g = jnp.concatenate(
        [edge_index[1], jnp.zeros((pad,), jnp.int32)]).reshape(_NW, _NCH, _CH)
    dst3s = jnp.concatenate(
        [edge_index[1], jnp.full((pad,), _N, jnp.int32)]).reshape(
            _NW, _NCH, _CH)
    color2 = node_color_rep.reshape(_N, 1)
    role2 = node_role.reshape(_N, 1)
    rel3 = jnp.concatenate(
        [edge_relation, jnp.zeros((pad,), jnp.int32)]).reshape(_NEB, _EB, 1)

    bcemb8 = _pad_rows(blk_color_emb, 8)
    bremb8 = _pad_rows(blk_role_emb, 8)

    def table_args(l):
        return (ew1[l, 0:128], ew1[l, 128:256], ew1[l, 272:304],
                _pad_rows(lyr_role_emb[l], 8), _pad_rows(lyr_color_emb[l], 8),
                _pad_rows(rel_emb[l], 16), ew1[l, 256:272],
                eb1[l].reshape(1, _H))

    nxh = jax.ShapeDtypeStruct((_N, _H), jnp.float32)
    h, a_tab, b_tab, relc = pl.pallas_call(
        _encode_body,
        out_shape=[nxh, nxh, nxh, jax.ShapeDtypeStruct((16, _H), jnp.float32)],
    )(scalars, color2, role2, w_in[0:128], w_in[128:136], w_in[136:144],
      b_in.reshape(1, _H), bcemb8, bremb8, *table_args(0))

    sc_gather, sc_scatter = _sc_kernels()
    out = None
    for l in range(3):
        pre0 = sc_gather(a_tab, b_tab, src3, dst3g)
        m = _edge_call(pre0, rel3, relc, ew2[l], eb2[l].reshape(1, _H))
        aggp = sc_scatter(m, dst3s)
        node_in = (h, aggp[0, :_N], aggp[1, :_N], color2, role2,
                   nw1[l, 0:128], nw1[l, 128:256], nw1[l, 256:272],
                   _pad_rows(lyr_role_emb[l], 8), _pad_rows(lyr_color_emb[l], 8),
                   nb1[l].reshape(1, _H), nw2[l], nb2[l].reshape(1, _H),
                   ln_g[l].reshape(1, _H), ln_b[l].reshape(1, _H))
        if l < 2:
            h, a_tab, b_tab, relc = pl.pallas_call(
                functools.partial(_node_body, False),
                out_shape=[nxh, nxh, nxh,
                           jax.ShapeDtypeStruct((16, _H), jnp.float32)],
            )(*node_in, *table_args(l + 1))
        else:
            out = pl.pallas_call(
                functools.partial(_node_body, True),
                out_shape=jax.ShapeDtypeStruct((_N, 64), jnp.float32),
            )(*node_in, w_out, b_out.reshape(1, 64))
    return out


# speedup vs baseline: 1.0870x; 1.0870x over previous
import functools

import jax
import jax.numpy as jnp
from jax import lax
from jax.experimental import pallas as pl
from jax.experimental.pallas import tpu as pltpu
from jax.experimental.pallas import tpu_sc as plsc

_N = 10000
_E = 320000
_EP = 327680
_EH = _EP // 2
_H = 128
_NW = 32
_CH = 128
_NCH = 40
_EPW = _NCH * _CH
_NP = 10240
_RPT = _NP // 16
_EB = 4096
_NEB = _EH // _EB
_NB = 2000
_NNB = _N // _NB


def _silu(x):
    return x * jax.nn.sigmoid(x)


def _onehot(idx_col, k):
    rows = idx_col.shape[0]
    return (lax.broadcasted_iota(jnp.int32, (rows, k), 1) == idx_col).astype(
        jnp.float32)


def _dot(a, b):
    return jnp.dot(a, b, preferred_element_type=jnp.float32)


def _tables(h, oh_r, oh_c, ws_ref, wd_ref, w32_ref, remb_ref, cemb_ref,
            relp_ref, wrel_ref, eb1_ref):
    w32 = w32_ref[...]
    a = (_dot(h, ws_ref[...])
         + _dot(oh_r, _dot(remb_ref[...], w32[0:8]))
         + _dot(oh_c, _dot(cemb_ref[...], w32[16:24])))
    b = (_dot(h, wd_ref[...])
         + _dot(oh_r, _dot(remb_ref[...], w32[8:16]))
         + _dot(oh_c, _dot(cemb_ref[...], w32[24:32]))
         + eb1_ref[...])
    relc = _dot(relp_ref[...], wrel_ref[...])
    return a, b, relc


def _encode_body(scalars_ref, color_ref, role_ref, w0_ref, wc_ref, wr_ref,
                 bin_ref, bcemb_ref, bremb_ref,
                 ws_ref, wd_ref, w32_ref, remb_ref, cemb_ref, relp_ref,
                 wrel_ref, eb1_ref,
                 h_ref, a_ref, b_ref, relc_ref):
    oh_c = _onehot(color_ref[...], 8)
    oh_r = _onehot(role_ref[...], 8)
    h = (_dot(scalars_ref[...], w0_ref[...])
         + _dot(oh_c, _dot(bcemb_ref[...], wc_ref[...]))
         + _dot(oh_r, _dot(bremb_ref[...], wr_ref[...]))
         + bin_ref[...])
    h_ref[...] = h
    a, b, relc = _tables(h, oh_r, oh_c, ws_ref, wd_ref, w32_ref, remb_ref,
                         cemb_ref, relp_ref, wrel_ref, eb1_ref)
    a_ref[...] = a
    b_ref[...] = b
    relc_ref[...] = relc


def _node_body(last, h_ref, agg0_ref, agg1_ref, agg2_ref, agg3_ref,
               color_ref, role_ref,
               nwa_ref, nwb_ref, nw16_ref, nremb_ref, ncemb_ref, nb1_ref,
               nw2_ref, nb2_ref, lng_ref, lnb_ref, *rest):
    oh_c = _onehot(color_ref[...], 8)
    oh_r = _onehot(role_ref[...], 8)
    h = h_ref[...]
    agg = ((agg0_ref[...] + agg1_ref[...])
           + (agg2_ref[...] + agg3_ref[...]))
    nw16 = nw16_ref[...]
    node_pre = (_dot(h, nwa_ref[...]) + _dot(agg, nwb_ref[...])
                + _dot(oh_r, _dot(nremb_ref[...], nw16[0:8]))
                + _dot(oh_c, _dot(ncemb_ref[...], nw16[8:16]))
                + nb1_ref[...])
    update = _dot(_silu(node_pre), nw2_ref[...]) + nb2_ref[...]
    hr = h + update
    mu = jnp.mean(hr, axis=-1, keepdims=True)
    d = hr - mu
    var = jnp.mean(d * d, axis=-1, keepdims=True)
    hn = d * lax.rsqrt(var + 1e-5) * lng_ref[...] + lnb_ref[...]
    if last:
        (wout_ref, bout_ref, out_ref) = rest
        out_ref[...] = _dot(hn, wout_ref[...]) + bout_ref[...]
    else:
        (ws_ref, wd_ref, w32_ref, remb_ref, cemb_ref, relp_ref, wrel_ref,
         eb1_ref, hn_ref, a_ref, b_ref, relc_ref) = rest
        hn_ref[...] = hn
        a, b, relc = _tables(hn, oh_r, oh_c, ws_ref, wd_ref, w32_ref,
                             remb_ref, cemb_ref, relp_ref, wrel_ref, eb1_ref)
        a_ref[...] = a
        b_ref[...] = b
        relc_ref[...] = relc


def _edge_body(pre0_ref, rel_ref, relc_ref, ew2_ref, eb2_ref, m_ref):
    rel = rel_ref[0]
    oh = (lax.broadcasted_iota(jnp.int32, (_EB, 16), 1) == rel).astype(
        jnp.float32)
    pre1 = pre0_ref[...] + _dot(oh, relc_ref[...])
    t2 = _dot(_silu(pre1), ew2_ref[...]) + eb2_ref[...]
    m_ref[...] = _silu(t2)


def _edge_call(pre0, rel3, relc, ew2, eb2):
    return pl.pallas_call(
        _edge_body,
        grid=(_NEB,),
        in_specs=[
            pl.BlockSpec((_EB, _H), lambda i: (i, 0)),
            pl.BlockSpec((1, _EB, 1), lambda i: (i, 0, 0)),
            pl.BlockSpec((16, _H), lambda i: (0, 0)),
            pl.BlockSpec((_H, _H), lambda i: (0, 0)),
            pl.BlockSpec((1, _H), lambda i: (0, 0)),
        ],
        out_specs=pl.BlockSpec((_EB, _H), lambda i: (i, 0)),
        out_shape=jax.ShapeDtypeStruct((_EH, _H), jnp.float32),
    )(pre0, rel3, relc, ew2, eb2)


@functools.lru_cache(maxsize=None)
def _sc_kernels():
    mesh = plsc.VectorSubcoreMesh(core_axis_name="c", subcore_axis_name="s")

    @functools.partial(
        pl.kernel,
        out_type=jax.ShapeDtypeStruct((_EH, _H), jnp.float32),
        mesh=mesh,
        scratch_types=[
            pltpu.VMEM((_NCH, _CH), jnp.int32),
            pltpu.VMEM((_NCH, _CH), jnp.int32),
            pltpu.VMEM((_CH, _H), jnp.float32),
            pltpu.VMEM((_CH, _H), jnp.float32),
            pltpu.VMEM((_CH, _H), jnp.float32),
            pltpu.VMEM((_CH, _H), jnp.float32),
            pltpu.SemaphoreType.DMA,
            pltpu.SemaphoreType.DMA,
            pltpu.SemaphoreType.DMA,
            pltpu.SemaphoreType.DMA,
        ],
    )
    def _sc_gather(a_hbm, b_hbm, src3_hbm, dst3_hbm, out_hbm,
                   idxs_v, idxd_v, bufa0, bufb0, bufa1, bufb1,
                   ga0, gb0, ga1, gb1):
        wid = lax.axis_index("s") * 2 + lax.axis_index("c")
        wbase = wid * _EPW
        pltpu.sync_copy(src3_hbm.at[wid], idxs_v)
        pltpu.sync_copy(dst3_hbm.at[wid], idxd_v)

        def issue(j, bufa, bufb, ga, gb):
            pltpu.async_copy(a_hbm.at[idxs_v.at[j]], bufa, ga)
            pltpu.async_copy(b_hbm.at[idxd_v.at[j]], bufb, gb)

        def wait(buf, sem):
            pltpu.make_async_copy(a_hbm.at[pl.ds(0, _CH)], buf, sem).wait()

        def vadd(dst_ref, src_ref):
            def row(r, carry):
                for p in range(8):
                    sl = pl.ds(p * 16, 16)
                    dst_ref[r, sl] = dst_ref[r, sl] + src_ref[r, sl]
                return carry

            lax.fori_loop(0, _CH, row, 0, unroll=False)

        def finish(j, bufa, bufb, ga, gb):
            wait(bufa, ga)
            wait(bufb, gb)
            vadd(bufa, bufb)
            pltpu.sync_copy(bufa, out_hbm.at[pl.ds(wbase + j * _CH, _CH)])

        issue(0, bufa0, bufb0, ga0, gb0)

        def body(i, carry):
            j0 = 2 * i
            issue(j0 + 1, bufa1, bufb1, ga1, gb1)
            finish(j0, bufa0, bufb0, ga0, gb0)

            @pl.when(i < _NCH // 2 - 1)
            def _():
                issue(j0 + 2, bufa0, bufb0, ga0, gb0)

            finish(j0 + 1, bufa1, bufb1, ga1, gb1)
            return carry

        lax.fori_loop(0, _NCH // 2, body, 0, unroll=False)

    @functools.partial(
        pl.kernel,
        out_type=jax.ShapeDtypeStruct((2, _NP, _H), jnp.float32),
        mesh=mesh,
        scratch_types=[
            pltpu.VMEM_SHARED((_NP, _H), jnp.float32),
            pltpu.VMEM((_NCH, _CH), jnp.int32),
            pltpu.VMEM((_CH, _H), jnp.float32),
            pltpu.VMEM((_CH, _H), jnp.float32),
            pltpu.SemaphoreType.DMA,
            pltpu.SemaphoreType.DMA,
        ],
    )
    def _sc_scatter(m_hbm, dst3_hbm, out_hbm, agg_sh, idx_v, m0, m1, r0, r1):
        cid = lax.axis_index("c")
        sid = lax.axis_index("s")
        wid = sid * 2 + cid
        wbase = wid * _EPW
        pltpu.sync_copy(dst3_hbm.at[wid], idx_v)

        def zrow(r, carry):
            for p in range(8):
                m0[r, pl.ds(p * 16, 16)] = jnp.zeros((16,), jnp.float32)
            return carry

        lax.fori_loop(0, _CH, zrow, 0, unroll=False)
        for k in range(5):
            pltpu.sync_copy(m0, agg_sh.at[pl.ds(sid * _RPT + k * _CH, _CH)])
        plsc.subcore_barrier()

        def issue(j, buf, sem):
            pltpu.async_copy(m_hbm.at[pl.ds(wbase + j * _CH, _CH)], buf, sem)

        def finish(j, buf, sem):
            pltpu.make_async_copy(m_hbm.at[pl.ds(0, _CH)], buf, sem).wait()
            pltpu.sync_copy(buf, agg_sh.at[idx_v.at[j]], add=True)

        issue(0, m0, r0)

        def body(i, carry):
            j0 = 2 * i
            issue(j0 + 1, m1, r1)
            finish(j0, m0, r0)

            @pl.when(i < _NCH // 2 - 1)
            def _():
                issue(j0 + 2, m0, r0)

            finish(j0 + 1, m1, r1)
            return carry

        lax.fori_loop(0, _NCH // 2, body, 0, unroll=False)
        plsc.subcore_barrier()
        pltpu.sync_copy(agg_sh.at[pl.ds(sid * _RPT, _RPT)],
                        out_hbm.at[cid, pl.ds(sid * _RPT, _RPT)])

    return _sc_gather, _sc_scatter


def _pad_rows(x, rows):
    return jnp.pad(x, ((0, rows - x.shape[0]), (0, 0)))


def kernel(scalars, blk_color_emb, blk_role_emb, w_in, b_in, rel_emb,
           lyr_role_emb, lyr_color_emb, ew1, eb1, ew2, eb2, nw1, nb1, nw2,
           nb2, ln_g, ln_b, w_out, b_out, edge_index, edge_relation,
           node_color_rep, node_role):
    pad = _EP - _E

    def halves(x, fill):
        xp = jnp.concatenate([x, jnp.full((pad,), fill, jnp.int32)])
        return (xp[:_EH].reshape(_NW, _NCH, _CH),
                xp[_EH:].reshape(_NW, _NCH, _CH))

    src3 = halves(edge_index[0], 0)
    dst3g = halves(edge_index[1], 0)
    dst3s = halves(edge_index[1], _N)
    relp = jnp.concatenate([edge_relation, jnp.zeros((pad,), jnp.int32)])
    rel3 = (relp[:_EH].reshape(_NEB, _EB, 1),
            relp[_EH:].reshape(_NEB, _EB, 1))
    color2 = node_color_rep.reshape(_N, 1)
    role2 = node_role.reshape(_N, 1)

    bcemb8 = _pad_rows(blk_color_emb, 8)
    bremb8 = _pad_rows(blk_role_emb, 8)

    def table_args(l):
        return (ew1[l, 0:128], ew1[l, 128:256], ew1[l, 272:304],
                _pad_rows(lyr_role_emb[l], 8), _pad_rows(lyr_color_emb[l], 8),
                _pad_rows(rel_emb[l], 16), ew1[l, 256:272],
                eb1[l].reshape(1, _H))

    nxh = jax.ShapeDtypeStruct((_N, _H), jnp.float32)
    relc_t = jax.ShapeDtypeStruct((16, _H), jnp.float32)
    nblk = pl.BlockSpec((_NB, _H), lambda i: (i, 0))
    iblk = pl.BlockSpec((_NB, 1), lambda i: (i, 0))

    def full(shape):
        return pl.BlockSpec(shape, lambda i: tuple(0 for _ in shape))

    tab_specs = [full((_H, _H)), full((_H, _H)), full((32, _H)), full((8, 8)),
                 full((8, 8)), full((16, 16)), full((16, _H)), full((1, _H))]
    h, a_tab, b_tab, relc = pl.pallas_call(
        _encode_body,
        grid=(_NNB,),
        in_specs=[nblk, iblk, iblk, full((_H, _H)), full((8, _H)),
                  full((8, _H)), full((1, _H)), full((8, 8)), full((8, 8)),
                  *tab_specs],
        out_specs=[nblk, nblk, nblk, full((16, _H))],
        out_shape=[nxh, nxh, nxh, relc_t],
    )(scalars, color2, role2, w_in[0:128], w_in[128:136], w_in[136:144],
      b_in.reshape(1, _H), bcemb8, bremb8, *table_args(0))

    sc_gather, sc_scatter = _sc_kernels()
    out = None
    for l in range(3):
        eb2l = eb2[l].reshape(1, _H)
        pre_h0 = sc_gather(a_tab, b_tab, src3[0], dst3g[0])
        m_h0 = _edge_call(pre_h0, rel3[0], relc, ew2[l], eb2l)
        pre_h1 = sc_gather(a_tab, b_tab, src3[1], dst3g[1])
        agg_h0 = sc_scatter(m_h0, dst3s[0])
        m_h1 = _edge_call(pre_h1, rel3[1], relc, ew2[l], eb2l)
        agg_h1 = sc_scatter(m_h1, dst3s[1])
        node_in = (h, agg_h0[0, :_N], agg_h0[1, :_N],
                   agg_h1[0, :_N], agg_h1[1, :_N], color2, role2,
                   nw1[l, 0:128], nw1[l, 128:256], nw1[l, 256:272],
                   _pad_rows(lyr_role_emb[l], 8), _pad_rows(lyr_color_emb[l], 8),
                   nb1[l].reshape(1, _H), nw2[l], nb2[l].reshape(1, _H),
                   ln_g[l].reshape(1, _H), ln_b[l].reshape(1, _H))
        node_specs = [nblk, nblk, nblk, nblk, nblk, iblk, iblk,
                      full((_H, _H)), full((_H, _H)), full((16, _H)),
                      full((8, 8)), full((8, 8)), full((1, _H)),
                      full((_H, _H)), full((1, _H)), full((1, _H)),
                      full((1, _H))]
        if l < 2:
            h, a_tab, b_tab, relc = pl.pallas_call(
                functools.partial(_node_body, False),
                grid=(_NNB,),
                in_specs=node_specs + tab_specs,
                out_specs=[nblk, nblk, nblk, full((16, _H))],
                out_shape=[nxh, nxh, nxh, relc_t],
            )(*node_in, *table_args(l + 1))
        else:
            out = pl.pallas_call(
                functools.partial(_node_body, True),
                grid=(_NNB,),
                in_specs=node_specs + [full((_H, 64)), full((1, 64))],
                out_specs=pl.BlockSpec((_NB, 64), lambda i: (i, 0)),
                out_shape=jax.ShapeDtypeStruct((_N, 64), jnp.float32),
            )(*node_in, w_out, b_out.reshape(1, 64))
    return out


# speedup vs baseline: 1.0979x; 1.0100x over previous
import functools

import jax
import jax.numpy as jnp
from jax import lax
from jax.experimental import pallas as pl
from jax.experimental.pallas import tpu as pltpu
from jax.experimental.pallas import tpu_sc as plsc

_N = 10000
_E = 320000
_EP = 327680
_EH = _EP // 2
_H = 128
_NW = 32
_CH = 128
_NCH = 40
_EPW = _NCH * _CH
_NP = 10240
_RPT = _NP // 16
_EB = 4096
_NEB = _EH // _EB
_NB = 2048
_NNB = _NP // _NB


def _silu(x):
    return x * jax.nn.sigmoid(x)


def _onehot(idx_col, k):
    rows = idx_col.shape[0]
    return (lax.broadcasted_iota(jnp.int32, (rows, k), 1) == idx_col).astype(
        jnp.float32)


def _dot(a, b):
    return jnp.dot(a, b, preferred_element_type=jnp.float32)


def _tables(h, oh_r, oh_c, ws_ref, wd_ref, w32_ref, remb_ref, cemb_ref,
            relp_ref, wrel_ref, eb1_ref):
    w32 = w32_ref[...]
    a = (_dot(h, ws_ref[...])
         + _dot(oh_r, _dot(remb_ref[...], w32[0:8]))
         + _dot(oh_c, _dot(cemb_ref[...], w32[16:24])))
    b = (_dot(h, wd_ref[...])
         + _dot(oh_r, _dot(remb_ref[...], w32[8:16]))
         + _dot(oh_c, _dot(cemb_ref[...], w32[24:32]))
         + eb1_ref[...])
    relc = _dot(relp_ref[...], wrel_ref[...])
    return a, b, relc


def _encode_body(scalars_ref, color_ref, role_ref, w0_ref, wc_ref, wr_ref,
                 bin_ref, bcemb_ref, bremb_ref,
                 ws_ref, wd_ref, w32_ref, remb_ref, cemb_ref, relp_ref,
                 wrel_ref, eb1_ref,
                 h_ref, a_ref, b_ref, relc_ref):
    oh_c = _onehot(color_ref[...], 8)
    oh_r = _onehot(role_ref[...], 8)
    h = (_dot(scalars_ref[...], w0_ref[...])
         + _dot(oh_c, _dot(bcemb_ref[...], wc_ref[...]))
         + _dot(oh_r, _dot(bremb_ref[...], wr_ref[...]))
         + bin_ref[...])
    h_ref[...] = h
    a, b, relc = _tables(h, oh_r, oh_c, ws_ref, wd_ref, w32_ref, remb_ref,
                         cemb_ref, relp_ref, wrel_ref, eb1_ref)
    a_ref[...] = a
    b_ref[...] = b
    relc_ref[...] = relc


def _node_body(last, h_ref, agg0_ref, agg1_ref, agg2_ref, agg3_ref,
               color_ref, role_ref,
               nwa_ref, nwb_ref, nw16_ref, nremb_ref, ncemb_ref, nb1_ref,
               nw2_ref, nb2_ref, lng_ref, lnb_ref, *rest):
    oh_c = _onehot(color_ref[...], 8)
    oh_r = _onehot(role_ref[...], 8)
    h = h_ref[...]
    agg = ((agg0_ref[...] + agg1_ref[...])
           + (agg2_ref[...] + agg3_ref[...]))
    nw16 = nw16_ref[...]
    node_pre = (_dot(h, nwa_ref[...]) + _dot(agg, nwb_ref[...])
                + _dot(oh_r, _dot(nremb_ref[...], nw16[0:8]))
                + _dot(oh_c, _dot(ncemb_ref[...], nw16[8:16]))
                + nb1_ref[...])
    update = _dot(_silu(node_pre), nw2_ref[...]) + nb2_ref[...]
    hr = h + update
    mu = jnp.mean(hr, axis=-1, keepdims=True)
    d = hr - mu
    var = jnp.mean(d * d, axis=-1, keepdims=True)
    hn = d * lax.rsqrt(var + 1e-5) * lng_ref[...] + lnb_ref[...]
    if last:
        (wout_ref, bout_ref, out_ref) = rest
        out_ref[...] = _dot(hn, wout_ref[...]) + bout_ref[...]
    else:
        (ws_ref, wd_ref, w32_ref, remb_ref, cemb_ref, relp_ref, wrel_ref,
         eb1_ref, hn_ref, a_ref, b_ref, relc_ref) = rest
        hn_ref[...] = hn
        a, b, relc = _tables(hn, oh_r, oh_c, ws_ref, wd_ref, w32_ref,
                             remb_ref, cemb_ref, relp_ref, wrel_ref, eb1_ref)
        a_ref[...] = a
        b_ref[...] = b
        relc_ref[...] = relc


def _edge_body(pre0_ref, rel_ref, relc_ref, ew2_ref, eb2_ref, m_ref):
    rel = rel_ref[0]
    oh = (lax.broadcasted_iota(jnp.int32, (_EB, 16), 1) == rel).astype(
        jnp.float32)
    pre1 = pre0_ref[...] + _dot(oh, relc_ref[...])
    t2 = _dot(_silu(pre1), ew2_ref[...]) + eb2_ref[...]
    m_ref[...] = _silu(t2)


def _edge_call(pre0, rel3, relc, ew2, eb2):
    return pl.pallas_call(
        _edge_body,
        grid=(_NEB,),
        in_specs=[
            pl.BlockSpec((_EB, _H), lambda i: (i, 0)),
            pl.BlockSpec((1, _EB, 1), lambda i: (i, 0, 0)),
            pl.BlockSpec((16, _H), lambda i: (0, 0)),
            pl.BlockSpec((_H, _H), lambda i: (0, 0)),
            pl.BlockSpec((1, _H), lambda i: (0, 0)),
        ],
        out_specs=pl.BlockSpec((_EB, _H), lambda i: (i, 0)),
        out_shape=jax.ShapeDtypeStruct((_EH, _H), jnp.float32),
    )(pre0, rel3, relc, ew2, eb2)


@functools.lru_cache(maxsize=None)
def _sc_kernels():
    mesh = plsc.VectorSubcoreMesh(core_axis_name="c", subcore_axis_name="s")

    @functools.partial(
        pl.kernel,
        out_type=jax.ShapeDtypeStruct((_EH, _H), jnp.float32),
        mesh=mesh,
        scratch_types=[
            pltpu.VMEM((_NCH, _CH), jnp.int32),
            pltpu.VMEM((_NCH, _CH), jnp.int32),
            pltpu.VMEM((_CH, _H), jnp.float32),
            pltpu.VMEM((_CH, _H), jnp.float32),
            pltpu.VMEM((_CH, _H), jnp.float32),
            pltpu.VMEM((_CH, _H), jnp.float32),
            pltpu.SemaphoreType.DMA,
            pltpu.SemaphoreType.DMA,
            pltpu.SemaphoreType.DMA,
            pltpu.SemaphoreType.DMA,
        ],
    )
    def _sc_gather(a_hbm, b_hbm, src3_hbm, dst3_hbm, out_hbm,
                   idxs_v, idxd_v, bufa0, bufb0, bufa1, bufb1,
                   ga0, gb0, ga1, gb1):
        wid = lax.axis_index("s") * 2 + lax.axis_index("c")
        wbase = wid * _EPW
        pltpu.sync_copy(src3_hbm.at[wid], idxs_v)
        pltpu.sync_copy(dst3_hbm.at[wid], idxd_v)

        def issue(j, bufa, bufb, ga, gb):
            pltpu.async_copy(a_hbm.at[idxs_v.at[j]], bufa, ga)
            pltpu.async_copy(b_hbm.at[idxd_v.at[j]], bufb, gb)

        def wait(buf, sem):
            pltpu.make_async_copy(a_hbm.at[pl.ds(0, _CH)], buf, sem).wait()

        def vadd(dst_ref, src_ref):
            def row(r, carry):
                for p in range(8):
                    sl = pl.ds(p * 16, 16)
                    dst_ref[r, sl] = dst_ref[r, sl] + src_ref[r, sl]
                return carry

            lax.fori_loop(0, _CH, row, 0, unroll=False)

        def finish(j, bufa, bufb, ga, gb):
            wait(bufa, ga)
            wait(bufb, gb)
            vadd(bufa, bufb)
            pltpu.sync_copy(bufa, out_hbm.at[pl.ds(wbase + j * _CH, _CH)])

        issue(0, bufa0, bufb0, ga0, gb0)

        def body(i, carry):
            j0 = 2 * i
            issue(j0 + 1, bufa1, bufb1, ga1, gb1)
            finish(j0, bufa0, bufb0, ga0, gb0)

            @pl.when(i < _NCH // 2 - 1)
            def _():
                issue(j0 + 2, bufa0, bufb0, ga0, gb0)

            finish(j0 + 1, bufa1, bufb1, ga1, gb1)
            return carry

        lax.fori_loop(0, _NCH // 2, body, 0, unroll=False)

    @functools.partial(
        pl.kernel,
        out_type=jax.ShapeDtypeStruct((2, _NP, _H), jnp.float32),
        mesh=mesh,
        scratch_types=[
            pltpu.VMEM_SHARED((_NP, _H), jnp.float32),
            pltpu.VMEM((_NCH, _CH), jnp.int32),
            pltpu.VMEM((_CH, _H), jnp.float32),
            pltpu.VMEM((_CH, _H), jnp.float32),
            pltpu.SemaphoreType.DMA,
            pltpu.SemaphoreType.DMA,
        ],
    )
    def _sc_scatter(m_hbm, dst3_hbm, out_hbm, agg_sh, idx_v, m0, m1, r0, r1):
        cid = lax.axis_index("c")
        sid = lax.axis_index("s")
        wid = sid * 2 + cid
        wbase = wid * _EPW
        pltpu.sync_copy(dst3_hbm.at[wid], idx_v)

        def zrow(r, carry):
            for p in range(8):
                m0[r, pl.ds(p * 16, 16)] = jnp.zeros((16,), jnp.float32)
            return carry

        lax.fori_loop(0, _CH, zrow, 0, unroll=False)
        for k in range(5):
            pltpu.sync_copy(m0, agg_sh.at[pl.ds(sid * _RPT + k * _CH, _CH)])
        plsc.subcore_barrier()

        def issue(j, buf, sem):
            pltpu.async_copy(m_hbm.at[pl.ds(wbase + j * _CH, _CH)], buf, sem)

        def finish(j, buf, sem):
            pltpu.make_async_copy(m_hbm.at[pl.ds(0, _CH)], buf, sem).wait()
            pltpu.sync_copy(buf, agg_sh.at[idx_v.at[j]], add=True)

        issue(0, m0, r0)

        def body(i, carry):
            j0 = 2 * i
            issue(j0 + 1, m1, r1)
            finish(j0, m0, r0)

            @pl.when(i < _NCH // 2 - 1)
            def _():
                issue(j0 + 2, m0, r0)

            finish(j0 + 1, m1, r1)
            return carry

        lax.fori_loop(0, _NCH // 2, body, 0, unroll=False)
        plsc.subcore_barrier()
        pltpu.sync_copy(agg_sh.at[pl.ds(sid * _RPT, _RPT)],
                        out_hbm.at[cid, pl.ds(sid * _RPT, _RPT)])

    return _sc_gather, _sc_scatter


def _pad_rows(x, rows):
    return jnp.pad(x, ((0, rows - x.shape[0]), (0, 0)))


def kernel(scalars, blk_color_emb, blk_role_emb, w_in, b_in, rel_emb,
           lyr_role_emb, lyr_color_emb, ew1, eb1, ew2, eb2, nw1, nb1, nw2,
           nb2, ln_g, ln_b, w_out, b_out, edge_index, edge_relation,
           node_color_rep, node_role):
    pad = _EP - _E

    def halves(x, fill):
        xp = jnp.concatenate([x, jnp.full((pad,), fill, jnp.int32)])
        return (xp[:_EH].reshape(_NW, _NCH, _CH),
                xp[_EH:].reshape(_NW, _NCH, _CH))

    src3 = halves(edge_index[0], 0)
    dst3g = halves(edge_index[1], 0)
    dst3s = halves(edge_index[1], _N)
    relp = jnp.concatenate([edge_relation, jnp.zeros((pad,), jnp.int32)])
    rel3 = (relp[:_EH].reshape(_NEB, _EB, 1),
            relp[_EH:].reshape(_NEB, _EB, 1))
    npad = _NP - _N
    scalars_p = jnp.pad(scalars, ((0, npad), (0, 0)))
    color2 = jnp.pad(node_color_rep, (0, npad)).reshape(_NP, 1)
    role2 = jnp.pad(node_role, (0, npad)).reshape(_NP, 1)

    bcemb8 = _pad_rows(blk_color_emb, 8)
    bremb8 = _pad_rows(blk_role_emb, 8)

    def table_args(l):
        return (ew1[l, 0:128], ew1[l, 128:256], ew1[l, 272:304],
                _pad_rows(lyr_role_emb[l], 8), _pad_rows(lyr_color_emb[l], 8),
                _pad_rows(rel_emb[l], 16), ew1[l, 256:272],
                eb1[l].reshape(1, _H))

    nxh = jax.ShapeDtypeStruct((_NP, _H), jnp.float32)
    relc_t = jax.ShapeDtypeStruct((16, _H), jnp.float32)
    nblk = pl.BlockSpec((_NB, _H), lambda i: (i, 0))
    iblk = pl.BlockSpec((_NB, 1), lambda i: (i, 0))

    def full(shape):
        return pl.BlockSpec(shape, lambda i: tuple(0 for _ in shape))

    tab_specs = [full((_H, _H)), full((_H, _H)), full((32, _H)), full((8, 8)),
                 full((8, 8)), full((16, 16)), full((16, _H)), full((1, _H))]
    h, a_tab, b_tab, relc = pl.pallas_call(
        _encode_body,
        grid=(_NNB,),
        in_specs=[nblk, iblk, iblk, full((_H, _H)), full((8, _H)),
                  full((8, _H)), full((1, _H)), full((8, 8)), full((8, 8)),
                  *tab_specs],
        out_specs=[nblk, nblk, nblk, full((16, _H))],
        out_shape=[nxh, nxh, nxh, relc_t],
    )(scalars_p, color2, role2, w_in[0:128], w_in[128:136], w_in[136:144],
      b_in.reshape(1, _H), bcemb8, bremb8, *table_args(0))

    sc_gather, sc_scatter = _sc_kernels()
    out = None
    for l in range(3):
        eb2l = eb2[l].reshape(1, _H)
        pre_h0 = sc_gather(a_tab, b_tab, src3[0], dst3g[0])
        m_h0 = _edge_call(pre_h0, rel3[0], relc, ew2[l], eb2l)
        pre_h1 = sc_gather(a_tab, b_tab, src3[1], dst3g[1])
        agg_h0 = sc_scatter(m_h0, dst3s[0])
        m_h1 = _edge_call(pre_h1, rel3[1], relc, ew2[l], eb2l)
        agg_h1 = sc_scatter(m_h1, dst3s[1])
        node_in = (h, agg_h0[0], agg_h0[1],
                   agg_h1[0], agg_h1[1], color2, role2,
                   nw1[l, 0:128], nw1[l, 128:256], nw1[l, 256:272],
                   _pad_rows(lyr_role_emb[l], 8), _pad_rows(lyr_color_emb[l], 8),
                   nb1[l].reshape(1, _H), nw2[l], nb2[l].reshape(1, _H),
                   ln_g[l].reshape(1, _H), ln_b[l].reshape(1, _H))
        node_specs = [nblk, nblk, nblk, nblk, nblk, iblk, iblk,
                      full((_H, _H)), full((_H, _H)), full((16, _H)),
                      full((8, 8)), full((8, 8)), full((1, _H)),
                      full((_H, _H)), full((1, _H)), full((1, _H)),
                      full((1, _H))]
        if l < 2:
            h, a_tab, b_tab, relc = pl.pallas_call(
                functools.partial(_node_body, False),
                grid=(_NNB,),
                in_specs=node_specs + tab_specs,
                out_specs=[nblk, nblk, nblk, full((16, _H))],
                out_shape=[nxh, nxh, nxh, relc_t],
            )(*node_in, *table_args(l + 1))
        else:
            out = pl.pallas_call(
                functools.partial(_node_body, True),
                grid=(_NNB,),
                in_specs=node_specs + [full((_H, 64)), full((1, 64))],
                out_specs=pl.BlockSpec((_NB, 64), lambda i: (i, 0)),
                out_shape=jax.ShapeDtypeStruct((_NP, 64), jnp.float32),
            )(*node_in, w_out, b_out.reshape(1, 64))
    return out[:_N]


# speedup vs baseline: 1.0996x; 1.0015x over previous
import functools

import jax
import jax.numpy as jnp
from jax import lax
from jax.experimental import pallas as pl
from jax.experimental.pallas import tpu as pltpu
from jax.experimental.pallas import tpu_sc as plsc

_N = 10000
_E = 320000
_EP = 327680
_EH = _EP // 2
_H = 128
_NW = 32
_CH = 128
_NCH = 40
_EPW = _NCH * _CH
_NP = 10240
_RPT = _NP // 16
_EB = 4096
_NEB = _EH // _EB
_NB = 2048
_NNB = _NP // _NB


def _silu(x):
    return x * jax.nn.sigmoid(x)


def _onehot(idx_col, k):
    rows = idx_col.shape[0]
    return (lax.broadcasted_iota(jnp.int32, (rows, k), 1) == idx_col).astype(
        jnp.float32)


def _dot(a, b):
    return jnp.dot(a, b, preferred_element_type=jnp.float32)


def _tables(h, oh_r, oh_c, ws_ref, wd_ref, w32_ref, remb_ref, cemb_ref,
            relp_ref, wrel_ref, eb1_ref):
    w32 = w32_ref[...]
    a = (_dot(h, ws_ref[...])
         + _dot(oh_r, _dot(remb_ref[...], w32[0:8]))
         + _dot(oh_c, _dot(cemb_ref[...], w32[16:24])))
    b = (_dot(h, wd_ref[...])
         + _dot(oh_r, _dot(remb_ref[...], w32[8:16]))
         + _dot(oh_c, _dot(cemb_ref[...], w32[24:32]))
         + eb1_ref[...])
    relc = _dot(relp_ref[...], wrel_ref[...])
    return a, b, relc


def _encode_body(scalars_ref, color_ref, role_ref, w0_ref, wc_ref, wr_ref,
                 bin_ref, bcemb_ref, bremb_ref,
                 ws_ref, wd_ref, w32_ref, remb_ref, cemb_ref, relp_ref,
                 wrel_ref, eb1_ref,
                 h_ref, a_ref, b_ref, relc_ref):
    oh_c = _onehot(color_ref[...], 8)
    oh_r = _onehot(role_ref[...], 8)
    h = (_dot(scalars_ref[...], w0_ref[...])
         + _dot(oh_c, _dot(bcemb_ref[...], wc_ref[...]))
         + _dot(oh_r, _dot(bremb_ref[...], wr_ref[...]))
         + bin_ref[...])
    h_ref[...] = h
    a, b, relc = _tables(h, oh_r, oh_c, ws_ref, wd_ref, w32_ref, remb_ref,
                         cemb_ref, relp_ref, wrel_ref, eb1_ref)
    a_ref[...] = a
    b_ref[...] = b
    relc_ref[...] = relc


def _node_body(last, h_ref, agg0_ref, agg1_ref, agg2_ref, agg3_ref,
               color_ref, role_ref,
               nwa_ref, nwb_ref, nw16_ref, nremb_ref, ncemb_ref, nb1_ref,
               nw2_ref, nb2_ref, lng_ref, lnb_ref, *rest):
    oh_c = _onehot(color_ref[...], 8)
    oh_r = _onehot(role_ref[...], 8)
    h = h_ref[...]
    agg = ((agg0_ref[...] + agg1_ref[...])
           + (agg2_ref[...] + agg3_ref[...]))
    nw16 = nw16_ref[...]
    node_pre = (_dot(h, nwa_ref[...]) + _dot(agg, nwb_ref[...])
                + _dot(oh_r, _dot(nremb_ref[...], nw16[0:8]))
                + _dot(oh_c, _dot(ncemb_ref[...], nw16[8:16]))
                + nb1_ref[...])
    update = _dot(_silu(node_pre), nw2_ref[...]) + nb2_ref[...]
    hr = h + update
    mu = jnp.mean(hr, axis=-1, keepdims=True)
    d = hr - mu
    var = jnp.mean(d * d, axis=-1, keepdims=True)
    hn = d * lax.rsqrt(var + 1e-5) * lng_ref[...] + lnb_ref[...]
    if last:
        (wout_ref, bout_ref, out_ref) = rest
        out_ref[...] = _dot(hn, wout_ref[...]) + bout_ref[...]
    else:
        (ws_ref, wd_ref, w32_ref, remb_ref, cemb_ref, relp_ref, wrel_ref,
         eb1_ref, hn_ref, a_ref, b_ref, relc_ref) = rest
        hn_ref[...] = hn
        a, b, relc = _tables(hn, oh_r, oh_c, ws_ref, wd_ref, w32_ref,
                             remb_ref, cemb_ref, relp_ref, wrel_ref, eb1_ref)
        a_ref[...] = a
        b_ref[...] = b
        relc_ref[...] = relc


def _edge_body(pre0_ref, rel_ref, relc_ref, ew2_ref, eb2_ref, m_ref):
    rel = rel_ref[0]
    oh = (lax.broadcasted_iota(jnp.int32, (_EB, 16), 1) == rel).astype(
        jnp.float32)
    pre1 = pre0_ref[...] + _dot(oh, relc_ref[...])
    t2 = _dot(_silu(pre1), ew2_ref[...]) + eb2_ref[...]
    m_ref[...] = _silu(t2)


def _edge_call(pre0, rel3, relc, ew2, eb2):
    return pl.pallas_call(
        _edge_body,
        grid=(_NEB,),
        in_specs=[
            pl.BlockSpec((_EB, _H), lambda i: (i, 0)),
            pl.BlockSpec((1, _EB, 1), lambda i: (i, 0, 0)),
            pl.BlockSpec((16, _H), lambda i: (0, 0)),
            pl.BlockSpec((_H, _H), lambda i: (0, 0)),
            pl.BlockSpec((1, _H), lambda i: (0, 0)),
        ],
        out_specs=pl.BlockSpec((_EB, _H), lambda i: (i, 0)),
        out_shape=jax.ShapeDtypeStruct((_EH, _H), jnp.float32),
    )(pre0, rel3, relc, ew2, eb2)


@functools.lru_cache(maxsize=None)
def _sc_kernels():
    mesh = plsc.VectorSubcoreMesh(core_axis_name="c", subcore_axis_name="s")

    @functools.partial(
        pl.kernel,
        out_type=jax.ShapeDtypeStruct((_EH, _H), jnp.float32),
        mesh=mesh,
        scratch_types=[
            pltpu.VMEM((_NCH, _CH), jnp.int32),
            pltpu.VMEM((_NCH, _CH), jnp.int32),
            pltpu.VMEM((_CH, _H), jnp.float32),
            pltpu.VMEM((_CH, _H), jnp.float32),
            pltpu.VMEM((_CH, _H), jnp.float32),
            pltpu.VMEM((_CH, _H), jnp.float32),
            pltpu.VMEM((_CH, _H), jnp.float32),
            pltpu.VMEM((_CH, _H), jnp.float32),
            pltpu.SemaphoreType.DMA,
            pltpu.SemaphoreType.DMA,
            pltpu.SemaphoreType.DMA,
            pltpu.SemaphoreType.DMA,
            pltpu.SemaphoreType.DMA,
            pltpu.SemaphoreType.DMA,
        ],
    )
    def _sc_gather(a_hbm, b_hbm, src3_hbm, dst3_hbm, out_hbm,
                   idxs_v, idxd_v, bufa0, bufb0, bufa1, bufb1, bufa2, bufb2,
                   ga0, gb0, ga1, gb1, ga2, gb2):
        wid = lax.axis_index("s") * 2 + lax.axis_index("c")
        wbase = wid * _EPW
        pltpu.sync_copy(src3_hbm.at[wid], idxs_v)
        pltpu.sync_copy(dst3_hbm.at[wid], idxd_v)
        slots = ((bufa0, bufb0, ga0, gb0), (bufa1, bufb1, ga1, gb1),
                 (bufa2, bufb2, ga2, gb2))

        def issue(j, slot):
            bufa, bufb, ga, gb = slot
            pltpu.async_copy(a_hbm.at[idxs_v.at[j]], bufa, ga)
            pltpu.async_copy(b_hbm.at[idxd_v.at[j]], bufb, gb)

        def vadd(dst_ref, src_ref):
            def row(r, carry):
                for p in range(8):
                    sl = pl.ds(p * 16, 16)
                    dst_ref[r, sl] = dst_ref[r, sl] + src_ref[r, sl]
                return carry

            lax.fori_loop(0, _CH, row, 0, unroll=False)

        def finish(j, slot):
            bufa, bufb, ga, gb = slot
            pltpu.make_async_copy(a_hbm.at[pl.ds(0, _CH)], bufa, ga).wait()
            pltpu.make_async_copy(a_hbm.at[pl.ds(0, _CH)], bufb, gb).wait()
            vadd(bufa, bufb)
            pltpu.sync_copy(bufa, out_hbm.at[pl.ds(wbase + j * _CH, _CH)])

        issue(0, slots[0])
        issue(1, slots[1])

        def body(i, carry):
            j0 = 3 * i
            issue(j0 + 2, slots[2])
            finish(j0, slots[0])
            issue(j0 + 3, slots[0])
            finish(j0 + 1, slots[1])

            @pl.when(i < _NCH // 3 - 1)
            def _():
                issue(j0 + 4, slots[1])

            finish(j0 + 2, slots[2])
            return carry

        lax.fori_loop(0, _NCH // 3, body, 0, unroll=False)
        finish(_NCH - 1, slots[0])

    @functools.partial(
        pl.kernel,
        out_type=jax.ShapeDtypeStruct((2, _NP, _H), jnp.float32),
        mesh=mesh,
        scratch_types=[
            pltpu.VMEM_SHARED((_NP, _H), jnp.float32),
            pltpu.VMEM((_NCH, _CH), jnp.int32),
            pltpu.VMEM((_CH, _H), jnp.float32),
            pltpu.VMEM((_CH, _H), jnp.float32),
            pltpu.SemaphoreType.DMA,
            pltpu.SemaphoreType.DMA,
        ],
    )
    def _sc_scatter(m_hbm, dst3_hbm, out_hbm, agg_sh, idx_v, m0, m1, r0, r1):
        cid = lax.axis_index("c")
        sid = lax.axis_index("s")
        wid = sid * 2 + cid
        wbase = wid * _EPW
        pltpu.sync_copy(dst3_hbm.at[wid], idx_v)

        def zrow(r, carry):
            for p in range(8):
                m0[r, pl.ds(p * 16, 16)] = jnp.zeros((16,), jnp.float32)
            return carry

        lax.fori_loop(0, _CH, zrow, 0, unroll=False)
        for k in range(5):
            pltpu.sync_copy(m0, agg_sh.at[pl.ds(sid * _RPT + k * _CH, _CH)])
        plsc.subcore_barrier()

        def issue(j, buf, sem):
            pltpu.async_copy(m_hbm.at[pl.ds(wbase + j * _CH, _CH)], buf, sem)

        def finish(j, buf, sem):
            pltpu.make_async_copy(m_hbm.at[pl.ds(0, _CH)], buf, sem).wait()
            pltpu.sync_copy(buf, agg_sh.at[idx_v.at[j]], add=True)

        issue(0, m0, r0)

        def body(i, carry):
            j0 = 2 * i
            issue(j0 + 1, m1, r1)
            finish(j0, m0, r0)

            @pl.when(i < _NCH // 2 - 1)
            def _():
                issue(j0 + 2, m0, r0)

            finish(j0 + 1, m1, r1)
            return carry

        lax.fori_loop(0, _NCH // 2, body, 0, unroll=False)
        plsc.subcore_barrier()
        pltpu.sync_copy(agg_sh.at[pl.ds(sid * _RPT, _RPT)],
                        out_hbm.at[cid, pl.ds(sid * _RPT, _RPT)])

    return _sc_gather, _sc_scatter


def _pad_rows(x, rows):
    return jnp.pad(x, ((0, rows - x.shape[0]), (0, 0)))


def kernel(scalars, blk_color_emb, blk_role_emb, w_in, b_in, rel_emb,
           lyr_role_emb, lyr_color_emb, ew1, eb1, ew2, eb2, nw1, nb1, nw2,
           nb2, ln_g, ln_b, w_out, b_out, edge_index, edge_relation,
           node_color_rep, node_role):
    pad = _EP - _E

    def halves(x, fill):
        xp = jnp.concatenate([x, jnp.full((pad,), fill, jnp.int32)])
        return (xp[:_EH].reshape(_NW, _NCH, _CH),
                xp[_EH:].reshape(_NW, _NCH, _CH))

    src3 = halves(edge_index[0], 0)
    dst3g = halves(edge_index[1], 0)
    dst3s = halves(edge_index[1], _N)
    relp = jnp.concatenate([edge_relation, jnp.zeros((pad,), jnp.int32)])
    rel3 = (relp[:_EH].reshape(_NEB, _EB, 1),
            relp[_EH:].reshape(_NEB, _EB, 1))
    npad = _NP - _N
    scalars_p = jnp.pad(scalars, ((0, npad), (0, 0)))
    color2 = jnp.pad(node_color_rep, (0, npad)).reshape(_NP, 1)
    role2 = jnp.pad(node_role, (0, npad)).reshape(_NP, 1)

    bcemb8 = _pad_rows(blk_color_emb, 8)
    bremb8 = _pad_rows(blk_role_emb, 8)

    def table_args(l):
        return (ew1[l, 0:128], ew1[l, 128:256], ew1[l, 272:304],
                _pad_rows(lyr_role_emb[l], 8), _pad_rows(lyr_color_emb[l], 8),
                _pad_rows(rel_emb[l], 16), ew1[l, 256:272],
                eb1[l].reshape(1, _H))

    nxh = jax.ShapeDtypeStruct((_NP, _H), jnp.float32)
    relc_t = jax.ShapeDtypeStruct((16, _H), jnp.float32)
    nblk = pl.BlockSpec((_NB, _H), lambda i: (i, 0))
    iblk = pl.BlockSpec((_NB, 1), lambda i: (i, 0))

    def full(shape):
        return pl.BlockSpec(shape, lambda i: tuple(0 for _ in shape))

    tab_specs = [full((_H, _H)), full((_H, _H)), full((32, _H)), full((8, 8)),
                 full((8, 8)), full((16, 16)), full((16, _H)), full((1, _H))]
    h, a_tab, b_tab, relc = pl.pallas_call(
        _encode_body,
        grid=(_NNB,),
        in_specs=[nblk, iblk, iblk, full((_H, _H)), full((8, _H)),
                  full((8, _H)), full((1, _H)), full((8, 8)), full((8, 8)),
                  *tab_specs],
        out_specs=[nblk, nblk, nblk, full((16, _H))],
        out_shape=[nxh, nxh, nxh, relc_t],
    )(scalars_p, color2, role2, w_in[0:128], w_in[128:136], w_in[136:144],
      b_in.reshape(1, _H), bcemb8, bremb8, *table_args(0))

    sc_gather, sc_scatter = _sc_kernels()
    out = None
    for l in range(3):
        eb2l = eb2[l].reshape(1, _H)
        pre_h0 = sc_gather(a_tab, b_tab, src3[0], dst3g[0])
        m_h0 = _edge_call(pre_h0, rel3[0], relc, ew2[l], eb2l)
        pre_h1 = sc_gather(a_tab, b_tab, src3[1], dst3g[1])
        agg_h0 = sc_scatter(m_h0, dst3s[0])
        m_h1 = _edge_call(pre_h1, rel3[1], relc, ew2[l], eb2l)
        agg_h1 = sc_scatter(m_h1, dst3s[1])
        node_in = (h, agg_h0[0], agg_h0[1],
                   agg_h1[0], agg_h1[1], color2, role2,
                   nw1[l, 0:128], nw1[l, 128:256], nw1[l, 256:272],
                   _pad_rows(lyr_role_emb[l], 8), _pad_rows(lyr_color_emb[l], 8),
                   nb1[l].reshape(1, _H), nw2[l], nb2[l].reshape(1, _H),
                   ln_g[l].reshape(1, _H), ln_b[l].reshape(1, _H))
        node_specs = [nblk, nblk, nblk, nblk, nblk, iblk, iblk,
                      full((_H, _H)), full((_H, _H)), full((16, _H)),
                      full((8, 8)), full((8, 8)), full((1, _H)),
                      full((_H, _H)), full((1, _H)), full((1, _H)),
                      full((1, _H))]
        if l < 2:
            h, a_tab, b_tab, relc = pl.pallas_call(
                functools.partial(_node_body, False),
                grid=(_NNB,),
                in_specs=node_specs + tab_specs,
                out_specs=[nblk, nblk, nblk, full((16, _H))],
                out_shape=[nxh, nxh, nxh, relc_t],
            )(*node_in, *table_args(l + 1))
        else:
            out = pl.pallas_call(
                functools.partial(_node_body, True),
                grid=(_NNB,),
                in_specs=node_specs + [full((_H, 64)), full((1, 64))],
                out_specs=pl.BlockSpec((_NB, 64), lambda i: (i, 0)),
                out_shape=jax.ShapeDtypeStruct((_NP, 64), jnp.float32),
            )(*node_in, w_out, b_out.reshape(1, 64))
    return out[:_N]
